# baseline (speedup 1.0000x reference)
import functools

import jax
import jax.numpy as jnp
from jax import lax
from jax.experimental import pallas as pl
from jax.experimental.pallas import tpu as pltpu
from jax.experimental.pallas import tpu_sc as plsc

NUM_CORES = 2
NUM_SUBCORES = 16
NW = NUM_CORES * NUM_SUBCORES
D = 128
L = 16
CHUNK = 256
NBUF = 2

B_TOTAL = 4096 * 200
BPW = B_TOTAL // NW
NCH = BPW // CHUNK

_mesh = plsc.VectorSubcoreMesh(core_axis_name="c", subcore_axis_name="s")


@functools.partial(
    pl.kernel,
    mesh=_mesh,
    compiler_params=pltpu.CompilerParams(needs_layout_passes=False),
    out_type=jax.ShapeDtypeStruct((B_TOTAL, D), jnp.float32),
    scratch_types=[
        pltpu.VMEM((BPW,), jnp.int32),
        pltpu.VMEM((3, D), jnp.float32),
        pltpu.VMEM((NBUF, CHUNK, D), jnp.float32),
        pltpu.SemaphoreType.DMA((NBUF,)),
    ],
)
def _lookup(table_hbm, idx_hbm, out_hbm, idx_v, tab_v, rows_v, ssem):
    wid = lax.axis_index("s") * NUM_CORES + lax.axis_index("c")
    base = wid * BPW

    pltpu.sync_copy(idx_hbm.at[pl.ds(base, BPW)], idx_v)
    pltpu.sync_copy(table_hbm, tab_v)

    trows = [[tab_v[k, pl.ds(L * j, L)] for j in range(D // L)]
             for k in range(3)]

    def scatter(i, b):
        return pltpu.make_async_copy(
            rows_v.at[b], out_hbm.at[pl.ds(base + i * CHUNK, CHUNK)],
            ssem.at[b])

    def compute_chunk(i, b):
        @plsc.parallel_loop(0, CHUNK, unroll=8)
        def row_body(r):
            g = i * CHUNK + r
            ie = plsc.load_gather(idx_v, [jnp.full((L,), g, jnp.int32)])
            m0 = ie == 0
            m1 = ie == 1
            for j in range(D // L):
                val = jnp.where(m0, trows[0][j],
                                jnp.where(m1, trows[1][j], trows[2][j]))
                rows_v[b, r, pl.ds(L * j, L)] = val

    def outer(k, carry):
        for b in range(NBUF):
            i = k * NBUF + b

            @pl.when(k > 0)
            def _():
                scatter(i - NBUF, b).wait()

            compute_chunk(i, b)
            scatter(i, b).start()
        return carry

    lax.fori_loop(0, NCH // NBUF, outer, 0)

    for b in range(NBUF):
        scatter(NCH - NBUF + b, b).wait()


def kernel(state, table):
    idx = state.reshape(B_TOTAL).astype(jnp.int32)
    out = _lookup(table, idx)
    return out.reshape(4096, 200, D)


# speedup vs baseline: 1.6442x; 1.6442x over previous
import functools

import jax
import jax.numpy as jnp
from jax import lax
from jax.experimental import pallas as pl
from jax.experimental.pallas import tpu as pltpu
from jax.experimental.pallas import tpu_sc as plsc

NUM_CORES = 2
NUM_SUBCORES = 16
NW = NUM_CORES * NUM_SUBCORES
D = 128
L = 16
CHUNK = 128
NBUF = 4

B_TOTAL = 4096 * 200
BPW = B_TOTAL // NW
NCH = BPW // CHUNK

_mesh = plsc.VectorSubcoreMesh(core_axis_name="c", subcore_axis_name="s")


@functools.partial(
    pl.kernel,
    mesh=_mesh,
    compiler_params=pltpu.CompilerParams(needs_layout_passes=False),
    out_type=jax.ShapeDtypeStruct((B_TOTAL, D), jnp.float32),
    scratch_types=[
        pltpu.VMEM((BPW,), jnp.int32),
        pltpu.VMEM((3, D), jnp.float32),
        pltpu.VMEM((NBUF, CHUNK, D), jnp.float32),
        pltpu.SemaphoreType.DMA((NBUF,)),
    ],
)
def _lookup(table_hbm, idx_hbm, out_hbm, idx_v, tab_v, rows_v, ssem):
    wid = lax.axis_index("s") * NUM_CORES + lax.axis_index("c")
    base = wid * BPW

    pltpu.sync_copy(idx_hbm.at[pl.ds(base, BPW)], idx_v)
    pltpu.sync_copy(table_hbm, tab_v)

    trows = [[tab_v[k, pl.ds(L * j, L)] for j in range(D // L)]
             for k in range(3)]

    def scatter(i, b):
        return pltpu.make_async_copy(
            rows_v.at[b], out_hbm.at[pl.ds(base + i * CHUNK, CHUNK)],
            ssem.at[b])

    def compute_chunk(i, b):
        @plsc.parallel_loop(0, CHUNK, unroll=4)
        def row_body(r):
            g = i * CHUNK + r
            ie = plsc.load_gather(idx_v, [jnp.full((L,), g, jnp.int32)])
            m0 = ie == 0
            m1 = ie == 1
            for j in range(D // L):
                val = jnp.where(m0, trows[0][j],
                                jnp.where(m1, trows[1][j], trows[2][j]))
                rows_v[b, r, pl.ds(L * j, L)] = val

    def outer(k, carry):
        for b in range(NBUF):
            i = k * NBUF + b

            @pl.when(k > 0)
            def _():
                scatter(i - NBUF, b).wait()

            compute_chunk(i, b)
            scatter(i, b).start()
        return carry

    lax.fori_loop(0, NCH // NBUF, outer, 0)

    for b in range(NBUF):
        scatter(NCH - NBUF + b, b).wait()


def kernel(state, table):
    idx = state.reshape(B_TOTAL).astype(jnp.int32)
    out = _lookup(table, idx)
    return out.reshape(4096, 200, D)


# speedup vs baseline: 1.6795x; 1.0215x over previous
import functools

import jax
import jax.numpy as jnp
from jax import lax
from jax.experimental import pallas as pl
from jax.experimental.pallas import tpu as pltpu
from jax.experimental.pallas import tpu_sc as plsc

NUM_CORES = 2
NUM_SUBCORES = 16
NW = NUM_CORES * NUM_SUBCORES
D = 128
L = 16
CHUNK = 320
NBUF = 2

B_TOTAL = 4096 * 200
BPW = B_TOTAL // NW
NCH = BPW // CHUNK

_mesh = plsc.VectorSubcoreMesh(core_axis_name="c", subcore_axis_name="s")


@functools.partial(
    pl.kernel,
    mesh=_mesh,
    compiler_params=pltpu.CompilerParams(needs_layout_passes=False),
    out_type=jax.ShapeDtypeStruct((B_TOTAL, D), jnp.float32),
    scratch_types=[
        pltpu.VMEM((BPW,), jnp.int32),
        pltpu.VMEM((3, D), jnp.float32),
        pltpu.VMEM((NBUF, CHUNK, D), jnp.float32),
        pltpu.SemaphoreType.DMA((NBUF,)),
    ],
)
def _lookup(table_hbm, idx_hbm, out_hbm, idx_v, tab_v, rows_v, ssem):
    wid = lax.axis_index("s") * NUM_CORES + lax.axis_index("c")
    base = wid * BPW

    pltpu.sync_copy(idx_hbm.at[pl.ds(base, BPW)], idx_v)
    pltpu.sync_copy(table_hbm, tab_v)

    trows = [[tab_v[k, pl.ds(L * j, L)] for j in range(D // L)]
             for k in range(3)]

    def scatter(i, b):
        return pltpu.make_async_copy(
            rows_v.at[b], out_hbm.at[pl.ds(base + i * CHUNK, CHUNK)],
            ssem.at[b])

    def compute_chunk(i, b):
        @plsc.parallel_loop(0, CHUNK, unroll=4)
        def row_body(r):
            g = i * CHUNK + r
            ie = plsc.load_gather(idx_v, [jnp.full((L,), g, jnp.int32)])
            m0 = ie == 0
            m1 = ie == 1
            for j in range(D // L):
                val = jnp.where(m0, trows[0][j],
                                jnp.where(m1, trows[1][j], trows[2][j]))
                rows_v[b, r, pl.ds(L * j, L)] = val

    def outer(k, carry):
        for b in range(NBUF):
            i = k * NBUF + b

            @pl.when(k > 0)
            def _():
                scatter(i - NBUF, b).wait()

            compute_chunk(i, b)
            scatter(i, b).start()
        return carry

    lax.fori_loop(0, NCH // NBUF, outer, 0)

    for b in range(NBUF):
        scatter(NCH - NBUF + b, b).wait()


def kernel(state, table):
    idx = state.reshape(B_TOTAL).astype(jnp.int32)
    out = _lookup(table, idx)
    return out.reshape(4096, 200, D)


# speedup vs baseline: 1.6848x; 1.0032x over previous
import functools

import jax
import jax.numpy as jnp
from jax import lax
from jax.experimental import pallas as pl
from jax.experimental.pallas import tpu as pltpu
from jax.experimental.pallas import tpu_sc as plsc

NUM_CORES = 2
NUM_SUBCORES = 16
NW = NUM_CORES * NUM_SUBCORES
D = 128
L = 16
CHUNK = 256
NBUF = 2

B_TOTAL = 4096 * 200
BPW = B_TOTAL // NW
NCH = BPW // CHUNK

_mesh = plsc.VectorSubcoreMesh(core_axis_name="c", subcore_axis_name="s")


@functools.partial(
    pl.kernel,
    mesh=_mesh,
    compiler_params=pltpu.CompilerParams(needs_layout_passes=False),
    out_type=jax.ShapeDtypeStruct((B_TOTAL, D), jnp.float32),
    scratch_types=[
        pltpu.VMEM((BPW,), jnp.int32),
        pltpu.VMEM((3, D), jnp.float32),
        pltpu.VMEM((NBUF, CHUNK, D), jnp.float32),
        pltpu.SemaphoreType.DMA((NBUF,)),
    ],
)
def _lookup(table_hbm, idx_hbm, out_hbm, idx_v, tab_v, rows_v, ssem):
    wid = lax.axis_index("s") * NUM_CORES + lax.axis_index("c")
    base = wid * BPW

    pltpu.sync_copy(idx_hbm.at[pl.ds(base, BPW)], idx_v)
    pltpu.sync_copy(table_hbm, tab_v)

    trows = [[tab_v[k, pl.ds(L * j, L)] for j in range(D // L)]
             for k in range(3)]

    def scatter(i, b):
        return pltpu.make_async_copy(
            rows_v.at[b], out_hbm.at[pl.ds(base + i * CHUNK, CHUNK)],
            ssem.at[b])

    def compute_chunk(i, b):
        @plsc.parallel_loop(0, CHUNK, unroll=4)
        def row_body(r):
            g = i * CHUNK + r
            ie = plsc.load_gather(idx_v, [jnp.full((L,), g, jnp.int32)])
            m0 = ie == 0
            m1 = ie == 1
            for j in range(D // L):
                val = jnp.where(m0, trows[0][j],
                                jnp.where(m1, trows[1][j], trows[2][j]))
                rows_v[b, r, pl.ds(L * j, L)] = val

    def outer(k, carry):
        for b in range(NBUF):
            i = k * NBUF + b

            @pl.when(k > 0)
            def _():
                scatter(i - NBUF, b).wait()

            compute_chunk(i, b)
            scatter(i, b).start()
        return carry

    lax.fori_loop(0, NCH // NBUF, outer, 0)

    for b in range(NBUF):
        scatter(NCH - NBUF + b, b).wait()


def kernel(state, table):
    idx = state.reshape(B_TOTAL).astype(jnp.int32)
    out = _lookup(table, idx)
    return out.reshape(4096, 200, D)
